# baseline (speedup 1.0000x reference)
import functools

import jax
import jax.numpy as jnp
from jax.experimental import pallas as pl
from jax.experimental.pallas import tpu as pltpu

_B, _N, _D, _K = 4, 4096, 128, 8
_TM = 512


def _threefry_hi0(x1, k0, k1):
    mask = 0xFFFFFFFF
    k2 = (k0 ^ k1 ^ 0x1BD11BDA) & mask
    ks = [k0, k1, k2]
    rot0 = (13, 15, 26, 6)
    rot1 = (17, 29, 16, 24)
    x0 = x1
    first = True
    for s in range(1, 6):
        for r in (rot0 if s % 2 == 1 else rot1):
            if first:
                first = False
            else:
                x0 = x0 + x1
            x1 = (x1 << jnp.uint32(r)) | (x1 >> jnp.uint32(32 - r))
            x1 = x1 ^ x0
        x0 = x0 + jnp.uint32(ks[s % 3])
        x1 = x1 + jnp.uint32((ks[(s + 1) % 3] + s) & mask)
    return x0, x1


def _body(scale_ref, xq_ref, xk_ref, sqk_ref, lidx_ref, lp_ref, idx_ref,
          *, n, k_top, tm):
    b = pl.program_id(0)
    ib = pl.program_id(1)
    qs = xq_ref[0]
    ks = xk_ref[0]
    sqk = sqk_ref[0]
    lidx = lidx_ref[...]
    scale = scale_ref[0, 0]

    inner = jax.lax.dot_general(
        qs, ks, (((1,), (1,)), ((), ())), preferred_element_type=jnp.float32
    )
    sqq = jnp.sum(qs * qs, axis=1, keepdims=True)
    d2 = (sqq + sqk) - 2.0 * inner
    d2 = jnp.maximum(d2, 0.0)
    logits = d2 * scale

    base = b.astype(jnp.uint32) * jnp.uint32(n * n) + (
        ib.astype(jnp.uint32) * jnp.uint32(tm * n)
    )
    o0, o1 = _threefry_hi0(lidx + base, 0, 1234)
    bits = o0 ^ o1
    u = jax.lax.bitcast_convert_type(
        (bits >> jnp.uint32(9)) | jnp.uint32(0x3F800000), jnp.float32
    ) - jnp.float32(1.0)
    q = u + jnp.float32(1e-8)
    vals = jnp.log(-jnp.log(q)) - logits

    col_f = jax.lax.broadcasted_iota(jnp.int32, (tm, n), 1).astype(jnp.float32)
    lps = []
    ids = []
    for j in range(k_top):
        m = jnp.max(vals, axis=1, keepdims=True)
        am = jnp.min(
            jnp.where(vals == m, col_f, jnp.float32(n)), axis=1, keepdims=True
        )
        lps.append(m)
        ids.append(am)
        if j < k_top - 1:
            vals = jnp.where(col_f == am, -jnp.inf, vals)
    lp_ref[0] = jnp.concatenate(lps, axis=1)
    idx_ref[0] = jnp.concatenate(ids, axis=1).astype(jnp.int32)


def _topk_call(x, scale, b, n, d, k_top, tm):
    sqk = jnp.sum(x * x, axis=-1)[:, None, :]
    lidx = (
        jnp.arange(tm, dtype=jnp.uint32)[:, None] * jnp.uint32(n)
        + jnp.arange(n, dtype=jnp.uint32)[None, :]
        + jnp.uint32(1234)
    )
    body = functools.partial(_body, n=n, k_top=k_top, tm=tm)
    return pl.pallas_call(
        body,
        grid=(b, n // tm),
        in_specs=[
            pl.BlockSpec(memory_space=pltpu.SMEM),
            pl.BlockSpec((1, tm, d), lambda bb, ii: (bb, ii, 0)),
            pl.BlockSpec((1, n, d), lambda bb, ii: (bb, 0, 0)),
            pl.BlockSpec((1, 1, n), lambda bb, ii: (bb, 0, 0)),
            pl.BlockSpec((tm, n), lambda bb, ii: (0, 0)),
        ],
        out_specs=[
            pl.BlockSpec((1, tm, k_top), lambda bb, ii: (bb, ii, 0)),
            pl.BlockSpec((1, tm, k_top), lambda bb, ii: (bb, ii, 0)),
        ],
        out_shape=[
            jax.ShapeDtypeStruct((b, n, k_top), jnp.float32),
            jax.ShapeDtypeStruct((b, n, k_top), jnp.int32),
        ],
        compiler_params=pltpu.CompilerParams(
            dimension_semantics=("parallel", "parallel")
        ),
    )(scale, x, x, sqk, lidx)


def kernel(x, A, temperature):
    scale = jnp.exp(jnp.clip(temperature, -5.0, 5.0)).reshape(1, 1)
    logprobs, indices = _topk_call(x, scale, _B, _N, _D, _K, _TM)
    rows = jnp.broadcast_to(
        jnp.arange(_N, dtype=indices.dtype)[None, :, None], (_B, _N, _K)
    )
    edges = jnp.stack(
        (indices.reshape(_B, -1), rows.reshape(_B, -1)), axis=-2
    )
    offsets = (jnp.arange(_B, dtype=indices.dtype) * _N)[:, None, None]
    edges_hat = jnp.transpose(edges + offsets, (1, 0, 2)).reshape(2, -1)
    return (x, edges_hat, logprobs)


# speedup vs baseline: 1.1935x; 1.1935x over previous
import functools

import jax
import jax.numpy as jnp
from jax.experimental import pallas as pl
from jax.experimental.pallas import tpu as pltpu

_B, _N, _D, _K = 4, 4096, 128, 8
_TM = 256


def _threefry_hi0(x1, k0, k1):
    mask = 0xFFFFFFFF
    k2 = (k0 ^ k1 ^ 0x1BD11BDA) & mask
    ks = [k0, k1, k2]
    rot0 = (13, 15, 26, 6)
    rot1 = (17, 29, 16, 24)
    x0 = x1
    first = True
    for s in range(1, 6):
        for r in (rot0 if s % 2 == 1 else rot1):
            if first:
                first = False
            else:
                x0 = x0 + x1
            x1 = (x1 << jnp.uint32(r)) | (x1 >> jnp.uint32(32 - r))
            x1 = x1 ^ x0
        x0 = x0 + jnp.uint32(ks[s % 3])
        x1 = x1 + jnp.uint32((ks[(s + 1) % 3] + s) & mask)
    return x0, x1


def _body(scale_ref, xq_ref, xk_ref, sqk_ref, lidx_ref, lp_ref, idx_ref,
          *, n, k_top, tm):
    b = pl.program_id(0)
    ib = pl.program_id(1)
    qs = xq_ref[0]
    ks = xk_ref[0]
    sqk = sqk_ref[0]
    lidx = lidx_ref[...]
    scale = scale_ref[0, 0]

    inner = jax.lax.dot_general(
        qs, ks, (((1,), (1,)), ((), ())), preferred_element_type=jnp.float32
    )
    sqq = jnp.sum(qs * qs, axis=1, keepdims=True)
    d2 = (sqq + sqk) - 2.0 * inner
    d2 = jnp.maximum(d2, 0.0)
    logits = d2 * scale

    base = b.astype(jnp.uint32) * jnp.uint32(n * n) + (
        ib.astype(jnp.uint32) * jnp.uint32(tm * n)
    )
    o0, o1 = _threefry_hi0(lidx + base, 0, 1234)
    bits = o0 ^ o1
    u = jax.lax.bitcast_convert_type(
        (bits >> jnp.uint32(9)) | jnp.uint32(0x3F800000), jnp.float32
    ) - jnp.float32(1.0)
    q = u + jnp.float32(1e-8)
    vals = jnp.log(-jnp.log(q)) - logits

    col_f = jax.lax.broadcasted_iota(jnp.int32, (tm, n), 1).astype(jnp.float32)
    lps = []
    ids = []
    for j in range(k_top):
        m = jnp.max(vals, axis=1, keepdims=True)
        am = jnp.min(
            jnp.where(vals == m, col_f, jnp.float32(n)), axis=1, keepdims=True
        )
        lps.append(m)
        ids.append(am)
        if j < k_top - 1:
            vals = jnp.where(col_f == am, -jnp.inf, vals)
    lp_ref[0] = jnp.concatenate(lps, axis=1)
    idx_ref[0] = jnp.concatenate(ids, axis=1).astype(jnp.int32)


def _topk_call(x, scale, b, n, d, k_top, tm):
    sqk = jnp.sum(x * x, axis=-1)[:, None, :]
    lidx = (
        jnp.arange(tm, dtype=jnp.uint32)[:, None] * jnp.uint32(n)
        + jnp.arange(n, dtype=jnp.uint32)[None, :]
        + jnp.uint32(1234)
    )
    body = functools.partial(_body, n=n, k_top=k_top, tm=tm)
    return pl.pallas_call(
        body,
        grid=(b, n // tm),
        in_specs=[
            pl.BlockSpec(memory_space=pltpu.SMEM),
            pl.BlockSpec((1, tm, d), lambda bb, ii: (bb, ii, 0)),
            pl.BlockSpec((1, n, d), lambda bb, ii: (bb, 0, 0)),
            pl.BlockSpec((1, 1, n), lambda bb, ii: (bb, 0, 0)),
            pl.BlockSpec((tm, n), lambda bb, ii: (0, 0)),
        ],
        out_specs=[
            pl.BlockSpec((1, tm, k_top), lambda bb, ii: (bb, ii, 0)),
            pl.BlockSpec((1, tm, k_top), lambda bb, ii: (bb, ii, 0)),
        ],
        out_shape=[
            jax.ShapeDtypeStruct((b, n, k_top), jnp.float32),
            jax.ShapeDtypeStruct((b, n, k_top), jnp.int32),
        ],
        compiler_params=pltpu.CompilerParams(
            dimension_semantics=("parallel", "parallel")
        ),
    )(scale, x, x, sqk, lidx)


def kernel(x, A, temperature):
    scale = jnp.exp(jnp.clip(temperature, -5.0, 5.0)).reshape(1, 1)
    logprobs, indices = _topk_call(x, scale, _B, _N, _D, _K, _TM)
    rows = jnp.broadcast_to(
        jnp.arange(_N, dtype=indices.dtype)[None, :, None], (_B, _N, _K)
    )
    edges = jnp.stack(
        (indices.reshape(_B, -1), rows.reshape(_B, -1)), axis=-2
    )
    offsets = (jnp.arange(_B, dtype=indices.dtype) * _N)[:, None, None]
    edges_hat = jnp.transpose(edges + offsets, (1, 0, 2)).reshape(2, -1)
    return (x, edges_hat, logprobs)
